# baseline (speedup 1.0000x reference)
import functools

import jax
import jax.numpy as jnp
from jax import lax
from jax.experimental import pallas as pl
from jax.experimental.pallas import tpu as pltpu
from jax.experimental.pallas import tpu_sc as plsc

_CHUNK = 512


@functools.partial(jax.jit, static_argnames=("n", "d"))
def _emb_lookup(x_flat, table, *, n, d):
    info = plsc.get_sparse_core_info()
    nw = info.num_cores * info.num_subcores
    per_w = n // nw
    steps = per_w // _CHUNK
    mesh = plsc.VectorSubcoreMesh(core_axis_name="c", subcore_axis_name="s")

    @functools.partial(
        pl.kernel,
        out_type=jax.ShapeDtypeStruct((n, d), jnp.float32),
        mesh=mesh,
        scratch_types=[
            pltpu.VMEM((_CHUNK,), jnp.int32),
            pltpu.VMEM((_CHUNK, d), jnp.float32),
            pltpu.SemaphoreType.DMA,
        ],
        compiler_params=pltpu.CompilerParams(use_tc_tiling_on_sc=False),
    )
    def emb(x_hbm, table_hbm, out_hbm, idx_v, rows_v, sem):
        wid = lax.axis_index("s") * info.num_cores + lax.axis_index("c")
        base = wid * per_w

        @pl.loop(0, steps)
        def _step(i):
            off = base + i * _CHUNK
            pltpu.sync_copy(x_hbm.at[pl.ds(off, _CHUNK)], idx_v)
            pltpu.async_copy(table_hbm.at[idx_v], rows_v, sem).wait()
            pltpu.sync_copy(rows_v, out_hbm.at[pl.ds(off, _CHUNK)])

    return emb(x_flat, table)


def kernel(x, table):
    b, l = x.shape
    _, d = table.shape
    n = b * l
    out = _emb_lookup(x.reshape(n).astype(jnp.int32), table, n=n, d=d)
    return out.reshape(b, l, d)


# speedup vs baseline: 1.0731x; 1.0731x over previous
import functools

import jax
import jax.numpy as jnp
from jax import lax
from jax.experimental import pallas as pl
from jax.experimental.pallas import tpu as pltpu
from jax.experimental.pallas import tpu_sc as plsc

_CHUNK = 800
_NBUF = 2


@functools.partial(jax.jit, static_argnames=("n", "d"))
def _emb_lookup(x_flat, table, *, n, d):
    info = plsc.get_sparse_core_info()
    nw = info.num_cores * info.num_subcores
    per_w = n // nw
    steps = per_w // _CHUNK
    mesh = plsc.VectorSubcoreMesh(core_axis_name="c", subcore_axis_name="s")

    @functools.partial(
        pl.kernel,
        out_type=jax.ShapeDtypeStruct((n, d), jnp.float32),
        mesh=mesh,
        scratch_types=[
            pltpu.VMEM((_NBUF, _CHUNK), jnp.int32),
            pltpu.VMEM((_NBUF, _CHUNK, d), jnp.float32),
            pltpu.SemaphoreType.DMA((_NBUF,)),
            pltpu.SemaphoreType.DMA((_NBUF,)),
            pltpu.SemaphoreType.DMA((_NBUF,)),
        ],
        compiler_params=pltpu.CompilerParams(use_tc_tiling_on_sc=False),
    )
    def emb(x_hbm, table_hbm, out_hbm, idx_v, rows_v, idx_sem, gat_sem, out_sem):
        wid = lax.axis_index("s") * info.num_cores + lax.axis_index("c")
        base = wid * per_w

        for b in range(_NBUF):
            pltpu.async_copy(
                x_hbm.at[pl.ds(base + b * _CHUNK, _CHUNK)], idx_v.at[b],
                idx_sem.at[b])

        @pl.loop(0, steps)
        def _step(i):
            b = lax.rem(i, _NBUF)
            off = base + i * _CHUNK

            @pl.when(i >= _NBUF)
            def _():
                pltpu.make_async_copy(
                    rows_v.at[b], out_hbm.at[pl.ds(0, _CHUNK)],
                    out_sem.at[b]).wait()

            pltpu.make_async_copy(
                x_hbm.at[pl.ds(off, _CHUNK)], idx_v.at[b],
                idx_sem.at[b]).wait()

            pltpu.async_copy(
                table_hbm.at[idx_v.at[b]], rows_v.at[b], gat_sem.at[b]).wait()

            pltpu.async_copy(
                rows_v.at[b], out_hbm.at[pl.ds(off, _CHUNK)], out_sem.at[b])

            @pl.when(i + _NBUF < steps)
            def _():
                pltpu.async_copy(
                    x_hbm.at[pl.ds(off + _NBUF * _CHUNK, _CHUNK)],
                    idx_v.at[b], idx_sem.at[b])

        for b in range(_NBUF):
            pltpu.make_async_copy(
                rows_v.at[b], out_hbm.at[pl.ds(0, _CHUNK)],
                out_sem.at[b]).wait()

    return emb(x_flat, table)


def kernel(x, table):
    b, l = x.shape
    _, d = table.shape
    n = b * l
    out = _emb_lookup(x.reshape(n).astype(jnp.int32), table, n=n, d=d)
    return out.reshape(b, l, d)
